# Initial kernel scaffold; baseline (speedup 1.0000x reference)
import jax
import jax.numpy as jnp
from jax.experimental import pallas as pl


def kernel(x, edge_index, batch, params):
    raise NotImplementedError("write your pallas kernel here")


# speedup vs baseline: 1.0647x; 1.0647x over previous
import functools

import jax
import jax.numpy as jnp
from jax.experimental import pallas as pl

N = 10000
E = 320000
DF = 128
H = 64
L = 3

_NODE_BLK = 1000
_EDGE_BLK = 3200


def _mm_kernel(x_ref, wt_ref, b_ref, o_ref, *, relu_in, relu_out):
    x = x_ref[...]
    if relu_in:
        x = jnp.maximum(x, 0.0)
    acc = jnp.dot(x, wt_ref[...], preferred_element_type=jnp.float32)
    acc = acc + b_ref[...]
    if relu_out:
        acc = jnp.maximum(acc, 0.0)
    o_ref[...] = acc


def _linear(x, wt, b, blk, relu_in=False, relu_out=False):
    m, k = x.shape
    n = wt.shape[1]
    grid = m // blk
    return pl.pallas_call(
        functools.partial(_mm_kernel, relu_in=relu_in, relu_out=relu_out),
        grid=(grid,),
        in_specs=[
            pl.BlockSpec((blk, k), lambda i: (i, 0)),
            pl.BlockSpec((k, n), lambda i: (0, 0)),
            pl.BlockSpec((1, n), lambda i: (0, 0)),
        ],
        out_specs=pl.BlockSpec((blk, n), lambda i: (i, 0)),
        out_shape=jax.ShapeDtypeStruct((m, n), jnp.float32),
    )(x, wt, b.reshape(1, n))


def kernel(x, edge_index, batch, params):
    src = edge_index[0]
    dst = edge_index[1]

    h = _linear(x, params['Wp'].T, params['bp'], _NODE_BLK, relu_out=True)

    for l in range(L):
        p = params['layers'][l]
        s = p['g'] * jax.lax.rsqrt(p['rv'] + 1e-5)
        t = p['be'] - p['rm'] * s
        W1a = p['W1'][:, :H]
        W1b = p['W1'][:, H:]
        Wa = (W1a - W1b) * s[:, None]
        Wb = W1b * s[:, None]
        c = p['b1'] * s + t

        A = _linear(h, Wa.T, c, _NODE_BLK)
        B = _linear(h, Wb.T, jnp.zeros((H,), jnp.float32), _NODE_BLK)

        P = A[dst] + B[src]
        Z = _linear(P, p['W2'].T, p['b2'], _EDGE_BLK, relu_in=True)

        agg = jax.ops.segment_max(Z, dst, num_segments=N)
        agg = jnp.maximum(jnp.where(jnp.isneginf(agg), 0.0, agg), 0.0)
        h = agg + h

    o = _linear(h, params['Wo1'].T, params['bo1'], _NODE_BLK, relu_out=True)
    o = _linear(o, params['Wo2'].T, params['bo2'], _NODE_BLK)
    return jnp.squeeze(o, -1)


# speedup vs baseline: 1.7420x; 1.6361x over previous
import functools

import jax
import jax.numpy as jnp
from jax import lax
from jax.experimental import pallas as pl
from jax.experimental.pallas import tpu as pltpu, tpu_sc as plsc

N = 10000
E = 320000
DF = 128
H = 64
L = 3

_NODE_BLK = 1000
_EDGE_BLK = 4000

_NC, _NS = 2, 16
_NW = _NC * _NS
_EPW = E // _NW
_CH = 80
_NCHUNK = _EPW // _CH


def _g_body(c_hbm, dst_hbm, src_hbm, p_hbm,
            idxd, idxs, bufD, bufS, bufP, semD, semS):
    wid = lax.axis_index("s") * _NC + lax.axis_index("c")
    base0 = wid * _EPW

    def chunk(j, carry):
        base = pl.multiple_of(base0 + j * _CH, _CH)
        base2 = pl.multiple_of(wid * (_EPW // 2) + j * (_CH // 2), _CH // 2)
        pltpu.sync_copy(dst_hbm.at[pl.ds(base, _CH)], idxd)
        pltpu.sync_copy(src_hbm.at[pl.ds(base, _CH)], idxs)
        cpD = pltpu.async_copy(c_hbm.at[idxd], bufD, semD)
        cpS = pltpu.async_copy(c_hbm.at[idxs], bufS, semS)
        cpD.wait()
        cpS.wait()

        def pair(i, c2):
            for q in range(4):
                lo = pl.ds(q * 16, 16)
                hi = pl.ds(64 + q * 16, 16)
                bufP[i, lo] = bufD[2 * i, lo] + bufS[2 * i, hi]
                bufP[i, hi] = bufD[2 * i + 1, lo] + bufS[2 * i + 1, hi]
            return c2

        lax.fori_loop(0, _CH // 2, pair, 0)
        pltpu.sync_copy(bufP, p_hbm.at[pl.ds(base2, _CH // 2)])
        return carry

    lax.fori_loop(0, _NCHUNK, chunk, 0)


def _gather_add(c, dst, src):
    mesh = plsc.VectorSubcoreMesh(core_axis_name="c", subcore_axis_name="s")
    return pl.kernel(
        _g_body,
        out_type=jax.ShapeDtypeStruct((E // 2, 2 * H), jnp.float32),
        mesh=mesh,
        scratch_types=[
            pltpu.VMEM((_CH,), jnp.int32),
            pltpu.VMEM((_CH,), jnp.int32),
            pltpu.VMEM((_CH, 2 * H), jnp.float32),
            pltpu.VMEM((_CH, 2 * H), jnp.float32),
            pltpu.VMEM((_CH // 2, 2 * H), jnp.float32),
            pltpu.SemaphoreType.DMA,
            pltpu.SemaphoreType.DMA,
        ],
    )(c, dst, src)


def _mm_kernel(x_ref, wt_ref, b_ref, o_ref, *, relu_in, relu_out):
    x = x_ref[...]
    if relu_in:
        x = jnp.maximum(x, 0.0)
    acc = jnp.dot(x, wt_ref[...], preferred_element_type=jnp.float32)
    acc = acc + b_ref[...]
    if relu_out:
        acc = jnp.maximum(acc, 0.0)
    o_ref[...] = acc


def _linear(x, wt, b, blk, relu_in=False, relu_out=False):
    m, k = x.shape
    n = wt.shape[1]
    grid = m // blk
    return pl.pallas_call(
        functools.partial(_mm_kernel, relu_in=relu_in, relu_out=relu_out),
        grid=(grid,),
        in_specs=[
            pl.BlockSpec((blk, k), lambda i: (i, 0)),
            pl.BlockSpec((k, n), lambda i: (0, 0)),
            pl.BlockSpec((1, n), lambda i: (0, 0)),
        ],
        out_specs=pl.BlockSpec((blk, n), lambda i: (i, 0)),
        out_shape=jax.ShapeDtypeStruct((m, n), jnp.float32),
    )(x, wt, b.reshape(1, n))


def kernel(x, edge_index, batch, params):
    src = edge_index[0]
    dst = edge_index[1]

    h = _linear(x, params['Wp'].T, params['bp'], _NODE_BLK, relu_out=True)

    for l in range(L):
        p = params['layers'][l]
        s = p['g'] * jax.lax.rsqrt(p['rv'] + 1e-5)
        t = p['be'] - p['rm'] * s
        W1a = p['W1'][:, :H]
        W1b = p['W1'][:, H:]
        Wa = (W1a - W1b) * s[:, None]
        Wb = W1b * s[:, None]
        c = p['b1'] * s + t

        wct = jnp.concatenate([Wa.T, Wb.T], axis=1)
        bc = jnp.concatenate([c, jnp.zeros((H,), jnp.float32)])
        C = _linear(h, wct, bc, _NODE_BLK)

        P = _gather_add(C, dst, src)

        w2t = p['W2'].T
        zeros = jnp.zeros_like(w2t)
        w2blk = jnp.block([[w2t, zeros], [zeros, w2t]])
        b22 = jnp.concatenate([p['b2'], p['b2']])
        Z2 = _linear(P, w2blk, b22, _EDGE_BLK, relu_in=True)

        Z = Z2.reshape(E, H)
        agg = jax.ops.segment_max(Z, dst, num_segments=N)
        agg = jnp.maximum(jnp.where(jnp.isneginf(agg), 0.0, agg), 0.0)
        h = agg + h

    o = _linear(h, params['Wo1'].T, params['bo1'], _NODE_BLK, relu_out=True)
    o = _linear(o, params['Wo2'].T, params['bo2'], _NODE_BLK)
    return jnp.squeeze(o, -1)
